# Initial kernel scaffold; baseline (speedup 1.0000x reference)
import jax
import jax.numpy as jnp
from jax.experimental import pallas as pl


def kernel(x, edge_index, mask, W1, b1, W2, b2):
    raise NotImplementedError("write your pallas kernel here")


# speedup vs baseline: 2.5040x; 2.5040x over previous
import functools

import jax
import jax.numpy as jnp
from jax import lax
from jax.experimental import pallas as pl
from jax.experimental.pallas import tpu as pltpu
from jax.experimental.pallas import tpu_sc as plsc

N = 10000
E = 320000
D_IN = 128
D_HID = 128
D_OUT = 64
DQ = 32

NC = 2
NS = 16
NW = NC * NS
CH = 128
GB = 4
N_CHUNK = 2560
E_PAD = N_CHUNK * CH
CPW = N_CHUNK // NW
DUMP = 10047
N_HIST = 10240
N_ACC = 2512
WBR = 160

_mesh2 = plsc.VectorSubcoreMesh(
    core_axis_name="c", subcore_axis_name="s", num_cores=NC, num_subcores=NS)


def _wb_base(s):
    return jnp.minimum(s * WBR, N_ACC - WBR)


@functools.partial(
    pl.kernel,
    out_type=jax.ShapeDtypeStruct((NW, N_HIST), jnp.float32),
    mesh=_mesh2,
    scratch_types=[
        pltpu.VMEM((GB, CH), jnp.int32),
        pltpu.VMEM((N_HIST,), jnp.float32),
    ],
    compiler_params=pltpu.CompilerParams(needs_layout_passes=False),
)
def _deg_kernel(dst2d, out, idx_v, hist_v):
    c = lax.axis_index("c")
    s = lax.axis_index("s")
    wid = s * NC + c

    @pl.loop(0, N_HIST // 16)
    def _(r):
        hist_v[pl.ds(r * 16, 16)] = jnp.zeros((16,), jnp.float32)

    ones = jnp.ones((16,), jnp.float32)
    wbase = wid * CPW

    @pl.loop(0, CPW // GB)
    def _(g):
        pltpu.sync_copy(dst2d.at[pl.ds(wbase + g * GB, GB)], idx_v)
        for b in range(GB):
            for j in range(CH // 16):
                d = idx_v[b, pl.ds(j * 16, 16)]
                plsc.addupdate_scatter(hist_v, [d], ones)

    pltpu.sync_copy(hist_v, out.at[wid])


def _make_agg(nphase):
  @functools.partial(
      pl.kernel,
      out_type=jax.ShapeDtypeStruct((nphase, NC, N_ACC, D_HID), jnp.float32),
      mesh=_mesh2,
      scratch_types=[
          pltpu.VMEM((GB, CH), jnp.int32),
          pltpu.VMEM((GB, CH), jnp.int32),
          pltpu.VMEM((GB, CH), jnp.int32),
          pltpu.VMEM((GB, CH), jnp.int32),
          pltpu.VMEM((GB, CH, D_HID), jnp.float32),
          pltpu.VMEM((WBR, D_HID), jnp.float32),
          pltpu.VMEM_SHARED((N_ACC, D_HID), jnp.float32),
          pltpu.SemaphoreType.DMA,
      ],
      compiler_params=pltpu.CompilerParams(needs_layout_passes=False),
  )
  def _agg(src2d, dst2d, *rest):
    tables = rest[:nphase]
    parts = rest[nphase]
    sidx_v, didx_v, gsrc_v, gdst_v, rows_v, wb_v, acc, sem = rest[nphase + 1:]
    c = lax.axis_index("c")
    s = lax.axis_index("s")
    wid = s * NC + c
    base = _wb_base(s)
    wbase = wid * CPW

    def fill_zeros():
        @pl.loop(0, WBR)
        def _(r):
            for j in range(D_HID // 16):
                wb_v[r, pl.ds(j * 16, 16)] = jnp.zeros((16,), jnp.float32)

    fill_zeros()
    for p in range(nphase):
        pltpu.sync_copy(wb_v, acc.at[pl.ds(base, WBR)])
        plsc.subcore_barrier()

        @pl.loop(0, CPW // GB)
        def _(g):
            gb = wbase + g * GB
            pltpu.sync_copy(src2d.at[pl.ds(gb, GB)], sidx_v)
            pltpu.sync_copy(dst2d.at[pl.ds(gb, GB)], didx_v)
            for b in range(GB):
                for j in range(CH // 16):
                    sl = pl.ds(j * 16, 16)
                    sv = sidx_v[b, sl]
                    dv = didx_v[b, sl]
                    gsrc_v[b, sl] = sv + (dv & 3) * N
                    gdst_v[b, sl] = lax.shift_right_logical(dv, 2)
            descs = [
                pltpu.async_copy(tables[p].at[gsrc_v.at[b]], rows_v.at[b],
                                 sem)
                for b in range(GB)
            ]
            for d in descs:
                d.wait()
            for b in range(GB):
                pltpu.sync_copy(rows_v.at[b], acc.at[gdst_v.at[b]], add=True)

        plsc.subcore_barrier()
        pltpu.sync_copy(acc.at[pl.ds(base, WBR)], wb_v)
        pltpu.sync_copy(wb_v, parts.at[p, c, pl.ds(base, WBR)])
        if p + 1 < nphase:
            fill_zeros()
            plsc.subcore_barrier()

  return _agg


_agg1 = _make_agg(4)
_agg2 = _make_agg(2)

_BN = 400
_GRID = N // _BN
_DC = 1280


def _tc0_body(d_b, o_b):
    ones = jnp.ones((NW, D_HID), jnp.float32)
    deg = lax.dot_general(d_b[...], ones, (((0,), (0,)), ((), ())),
                          preferred_element_type=jnp.float32)
    o_b[...] = 1.0 / jnp.sqrt(deg + 1.0)


_tc0 = pl.pallas_call(
    _tc0_body,
    grid=(N_HIST // _DC,),
    in_specs=[pl.BlockSpec((NW, _DC), lambda i: (0, i))],
    out_specs=pl.BlockSpec((_DC, D_HID), lambda i: (i, 0)),
    out_shape=jax.ShapeDtypeStruct((N_HIST, D_HID), jnp.float32),
)


def _tc1_body(x_b, w_b, d_b, o_b):
    h = jnp.dot(x_b[...], w_b[...], preferred_element_type=jnp.float32)
    o_b[...] = h * d_b[...]


def _tc2_body(a_b, h_b, d_b, d64_b, w_b, b1_b, o_b):
    pre = (a_b[0] + a_b[1] + h_b[...]) * d_b[...] + b1_b[...]
    h2 = jnp.maximum(pre, 0.0)
    o_b[...] = jnp.dot(
        h2, w_b[...], preferred_element_type=jnp.float32) * d64_b[...]


def _tc3_body(a_b, h_b, d64_b, b2_b, o_b):
    o = (a_b[0] + a_b[1] + h_b[...]) * d64_b[...] + b2_b[...]
    m = jnp.max(o, axis=1, keepdims=True)
    lse = jnp.log(jnp.sum(jnp.exp(o - m), axis=1, keepdims=True)) + m
    o_b[...] = o - lse


def _row_spec(d):
    return pl.BlockSpec((_BN, d), lambda i: (i, 0))


def _agg_spec(d):
    return pl.BlockSpec((NC, _BN, d), lambda i: (0, i, 0))


def _full_spec(shape):
    nd = len(shape)
    return pl.BlockSpec(shape, lambda i: (0,) * nd)


_tc1 = pl.pallas_call(
    _tc1_body,
    grid=(_GRID,),
    in_specs=[_row_spec(D_IN), _full_spec((D_IN, D_HID)), _row_spec(D_HID)],
    out_specs=_row_spec(D_HID),
    out_shape=jax.ShapeDtypeStruct((N, D_HID), jnp.float32),
)

_tc2 = pl.pallas_call(
    _tc2_body,
    grid=(_GRID,),
    in_specs=[
        _agg_spec(D_HID), _row_spec(D_HID), _row_spec(D_HID),
        _row_spec(D_OUT),
        _full_spec((D_HID, D_OUT)), _full_spec((1, D_HID)),
    ],
    out_specs=_row_spec(D_OUT),
    out_shape=jax.ShapeDtypeStruct((N, D_OUT), jnp.float32),
)

_tc3 = pl.pallas_call(
    _tc3_body,
    grid=(_GRID,),
    in_specs=[
        _agg_spec(D_OUT), _row_spec(D_OUT), _row_spec(D_OUT),
        _full_spec((1, D_OUT)),
    ],
    out_specs=_row_spec(D_OUT),
    out_shape=jax.ShapeDtypeStruct((N, D_OUT), jnp.float32),
)


def _stack_tables(h, nq):
    tabs = []
    for q in range(nq):
        hq = h[:, q * DQ:(q + 1) * DQ]
        planes = jnp.zeros((4, N, D_HID), jnp.float32)
        for k in range(4):
            planes = jax.lax.dynamic_update_slice(
                planes, hq[None], (k, 0, k * DQ))
        tabs.append(planes.reshape(4 * N, D_HID))
    return tabs


def _unpack_parts(parts, nq):
    pr = parts.reshape(nq, NC, 4 * N_ACC, DQ)
    return jnp.concatenate([pr[q] for q in range(nq)], axis=-1)


def kernel(x, edge_index, mask, W1, b1, W2, b2):
    del mask
    src = edge_index[0]
    dst = edge_index[1]
    pad = E_PAD - E
    src2d = jnp.concatenate([src, jnp.zeros((pad,), jnp.int32)]).reshape(
        N_CHUNK, CH)
    dst2d = jnp.concatenate([dst, jnp.full((pad,), DUMP, jnp.int32)]).reshape(
        N_CHUNK, CH)

    deg_t = _deg_kernel(dst2d)
    dis2d = _tc0(deg_t)
    dis64 = dis2d[:, :D_OUT]
    h1p = _tc1(x, W1, dis2d)
    parts1 = _agg1(src2d, dst2d, *_stack_tables(h1p, 4))
    agg1t = _unpack_parts(parts1, 4)
    h2w = _tc2(agg1t, h1p, dis2d, dis64, W2, b1.reshape(1, D_HID))
    parts2 = _agg2(src2d, dst2d, *_stack_tables(h2w, 2))
    agg2t = _unpack_parts(parts2, 2)
    return _tc3(agg2t, h2w, dis64, b2.reshape(1, D_OUT))


# speedup vs baseline: 2.7164x; 1.0848x over previous
import functools

import jax
import jax.numpy as jnp
from jax import lax
from jax.experimental import pallas as pl
from jax.experimental.pallas import tpu as pltpu
from jax.experimental.pallas import tpu_sc as plsc

N = 10000
E = 320000
D_IN = 128
D_HID = 128
D_OUT = 64
DQ = 32

NC = 2
NS = 16
NW = NC * NS
CH = 128
GB = 4
N_CHUNK = 2560
E_PAD = N_CHUNK * CH
CPW = N_CHUNK // NW
DUMP = 10047
N_HIST = 10240
N_ACC = 2512
WBR = 160

_mesh2 = plsc.VectorSubcoreMesh(
    core_axis_name="c", subcore_axis_name="s", num_cores=NC, num_subcores=NS)


def _wb_base(s):
    return jnp.minimum(s * WBR, N_ACC - WBR)


@functools.partial(
    pl.kernel,
    out_type=jax.ShapeDtypeStruct((NW, N_HIST), jnp.float32),
    mesh=_mesh2,
    scratch_types=[
        pltpu.VMEM((GB, CH), jnp.int32),
        pltpu.VMEM((N_HIST,), jnp.float32),
    ],
    compiler_params=pltpu.CompilerParams(needs_layout_passes=False),
)
def _deg_kernel(dst2d, out, idx_v, hist_v):
    c = lax.axis_index("c")
    s = lax.axis_index("s")
    wid = s * NC + c

    @pl.loop(0, N_HIST // 16)
    def _(r):
        hist_v[pl.ds(r * 16, 16)] = jnp.zeros((16,), jnp.float32)

    ones = jnp.ones((16,), jnp.float32)
    wbase = wid * CPW

    @pl.loop(0, CPW // GB)
    def _(g):
        pltpu.sync_copy(dst2d.at[pl.ds(wbase + g * GB, GB)], idx_v)
        for b in range(GB):
            for j in range(CH // 16):
                d = idx_v[b, pl.ds(j * 16, 16)]
                plsc.addupdate_scatter(hist_v, [d], ones)

    pltpu.sync_copy(hist_v, out.at[wid])


GBP = 2
STEPS = CPW // GBP


def _make_agg(nphase):
  @functools.partial(
      pl.kernel,
      out_type=jax.ShapeDtypeStruct((nphase, NC, N_ACC, D_HID), jnp.float32),
      mesh=_mesh2,
      scratch_types=[
          pltpu.VMEM((GB, CH), jnp.int32),
          pltpu.VMEM((GB, CH), jnp.int32),
          pltpu.VMEM((CPW, CH), jnp.int32),
          pltpu.VMEM((CPW, CH), jnp.int32),
          pltpu.VMEM((2, GBP, CH, D_HID), jnp.float32),
          pltpu.VMEM((WBR, D_HID), jnp.float32),
          pltpu.VMEM_SHARED((N_ACC, D_HID), jnp.float32),
          pltpu.SemaphoreType.DMA,
          pltpu.SemaphoreType.DMA,
          pltpu.SemaphoreType.DMA,
          pltpu.SemaphoreType.DMA,
      ],
      compiler_params=pltpu.CompilerParams(needs_layout_passes=False),
  )
  def _agg(src2d, dst2d, *rest):
    tables = rest[:nphase]
    parts = rest[nphase]
    (sidx_v, didx_v, gsrc_a, gdst_a, rbuf, wb_v, acc,
     sg0, sg1, ss0, ss1) = rest[nphase + 1:]
    sg = (sg0, sg1)
    ss = (ss0, ss1)
    c = lax.axis_index("c")
    s = lax.axis_index("s")
    wid = s * NC + c
    base = _wb_base(s)
    wbase = wid * CPW

    @pl.loop(0, CPW // GB)
    def _(g):
        pltpu.sync_copy(src2d.at[pl.ds(wbase + g * GB, GB)], sidx_v)
        pltpu.sync_copy(dst2d.at[pl.ds(wbase + g * GB, GB)], didx_v)
        for b in range(GB):
            for j in range(CH // 16):
                sl = pl.ds(j * 16, 16)
                sv = sidx_v[b, sl]
                dv = didx_v[b, sl]
                gsrc_a[g * GB + b, sl] = sv + (dv & 3) * N
                gdst_a[g * GB + b, sl] = lax.shift_right_logical(dv, 2)

    def fill_zeros():
        @pl.loop(0, WBR)
        def _(r):
            for j in range(D_HID // 16):
                wb_v[r, pl.ds(j * 16, 16)] = jnp.zeros((16,), jnp.float32)

    fill_zeros()
    for p in range(nphase):
        pltpu.sync_copy(wb_v, acc.at[pl.ds(base, WBR)])
        plsc.subcore_barrier()

        def fire_gathers(t, buf):
            for b in range(GBP):
                pltpu.async_copy(
                    tables[p].at[gsrc_a.at[t * GBP + b]],
                    rbuf.at[buf, b], sg[buf])

        def wait_gathers(buf):
            for b in range(GBP):
                pltpu.make_async_copy(
                    tables[p].at[gsrc_a.at[0]], rbuf.at[buf, b],
                    sg[buf]).wait()

        def fire_scatters(t, buf):
            for b in range(GBP):
                pltpu.async_copy(
                    rbuf.at[buf, b], acc.at[gdst_a.at[t * GBP + b]],
                    ss[buf], add=True)

        def wait_scatters(buf):
            for b in range(GBP):
                pltpu.make_async_copy(
                    rbuf.at[buf, b], acc.at[gdst_a.at[0]], ss[buf]).wait()

        fire_gathers(0, 0)

        @pl.loop(0, STEPS)
        def _(t):
            def body(buf):
                nxt = 1 - buf

                @pl.when(t + 1 < STEPS)
                def _():
                    @pl.when(t >= 1)
                    def _():
                        wait_scatters(nxt)

                    fire_gathers(t + 1, nxt)

                wait_gathers(buf)
                fire_scatters(t, buf)

            @pl.when(t % 2 == 0)
            def _():
                body(0)

            @pl.when(t % 2 == 1)
            def _():
                body(1)

        wait_scatters(0)
        wait_scatters(1)
        plsc.subcore_barrier()
        pltpu.sync_copy(acc.at[pl.ds(base, WBR)], wb_v)
        pltpu.sync_copy(wb_v, parts.at[p, c, pl.ds(base, WBR)])
        if p + 1 < nphase:
            fill_zeros()
            plsc.subcore_barrier()

  return _agg


_agg1 = _make_agg(4)
_agg2 = _make_agg(2)

_BN = 400
_GRID = N // _BN
_DC = 1280


def _tc0_body(d_b, o_b):
    ones = jnp.ones((NW, D_HID), jnp.float32)
    deg = lax.dot_general(d_b[...], ones, (((0,), (0,)), ((), ())),
                          preferred_element_type=jnp.float32)
    o_b[...] = 1.0 / jnp.sqrt(deg + 1.0)


_tc0 = pl.pallas_call(
    _tc0_body,
    grid=(N_HIST // _DC,),
    in_specs=[pl.BlockSpec((NW, _DC), lambda i: (0, i))],
    out_specs=pl.BlockSpec((_DC, D_HID), lambda i: (i, 0)),
    out_shape=jax.ShapeDtypeStruct((N_HIST, D_HID), jnp.float32),
)


def _tc1_body(x_b, w_b, d_b, o_b):
    h = jnp.dot(x_b[...], w_b[...], preferred_element_type=jnp.float32)
    o_b[...] = h * d_b[...]


def _tc2_body(a_b, h_b, d_b, d64_b, w_b, b1_b, o_b):
    pre = (a_b[0] + a_b[1] + h_b[...]) * d_b[...] + b1_b[...]
    h2 = jnp.maximum(pre, 0.0)
    o_b[...] = jnp.dot(
        h2, w_b[...], preferred_element_type=jnp.float32) * d64_b[...]


def _tc3_body(a_b, h_b, d64_b, b2_b, o_b):
    o = (a_b[0] + a_b[1] + h_b[...]) * d64_b[...] + b2_b[...]
    m = jnp.max(o, axis=1, keepdims=True)
    lse = jnp.log(jnp.sum(jnp.exp(o - m), axis=1, keepdims=True)) + m
    o_b[...] = o - lse


def _row_spec(d):
    return pl.BlockSpec((_BN, d), lambda i: (i, 0))


def _agg_spec(d):
    return pl.BlockSpec((NC, _BN, d), lambda i: (0, i, 0))


def _full_spec(shape):
    nd = len(shape)
    return pl.BlockSpec(shape, lambda i: (0,) * nd)


_tc1 = pl.pallas_call(
    _tc1_body,
    grid=(_GRID,),
    in_specs=[_row_spec(D_IN), _full_spec((D_IN, D_HID)), _row_spec(D_HID)],
    out_specs=_row_spec(D_HID),
    out_shape=jax.ShapeDtypeStruct((N, D_HID), jnp.float32),
)

_tc2 = pl.pallas_call(
    _tc2_body,
    grid=(_GRID,),
    in_specs=[
        _agg_spec(D_HID), _row_spec(D_HID), _row_spec(D_HID),
        _row_spec(D_OUT),
        _full_spec((D_HID, D_OUT)), _full_spec((1, D_HID)),
    ],
    out_specs=_row_spec(D_OUT),
    out_shape=jax.ShapeDtypeStruct((N, D_OUT), jnp.float32),
)

_tc3 = pl.pallas_call(
    _tc3_body,
    grid=(_GRID,),
    in_specs=[
        _agg_spec(D_OUT), _row_spec(D_OUT), _row_spec(D_OUT),
        _full_spec((1, D_OUT)),
    ],
    out_specs=_row_spec(D_OUT),
    out_shape=jax.ShapeDtypeStruct((N, D_OUT), jnp.float32),
)


def _stack_tables(h, nq):
    tabs = []
    for q in range(nq):
        hq = h[:, q * DQ:(q + 1) * DQ]
        planes = jnp.zeros((4, N, D_HID), jnp.float32)
        for k in range(4):
            planes = jax.lax.dynamic_update_slice(
                planes, hq[None], (k, 0, k * DQ))
        tabs.append(planes.reshape(4 * N, D_HID))
    return tabs


def _unpack_parts(parts, nq):
    pr = parts.reshape(nq, NC, 4 * N_ACC, DQ)
    return jnp.concatenate([pr[q] for q in range(nq)], axis=-1)


def kernel(x, edge_index, mask, W1, b1, W2, b2):
    del mask
    src = edge_index[0]
    dst = edge_index[1]
    pad = E_PAD - E
    src2d = jnp.concatenate([src, jnp.zeros((pad,), jnp.int32)]).reshape(
        N_CHUNK, CH)
    dst2d = jnp.concatenate([dst, jnp.full((pad,), DUMP, jnp.int32)]).reshape(
        N_CHUNK, CH)

    deg_t = _deg_kernel(dst2d)
    dis2d = _tc0(deg_t)
    dis64 = dis2d[:, :D_OUT]
    h1p = _tc1(x, W1, dis2d)
    parts1 = _agg1(src2d, dst2d, *_stack_tables(h1p, 4))
    agg1t = _unpack_parts(parts1, 4)
    h2w = _tc2(agg1t, h1p, dis2d, dis64, W2, b1.reshape(1, D_HID))
    parts2 = _agg2(src2d, dst2d, *_stack_tables(h2w, 2))
    agg2t = _unpack_parts(parts2, 2)
    return _tc3(agg2t, h2w, dis64, b2.reshape(1, D_OUT))


# speedup vs baseline: 2.7232x; 1.0025x over previous
import functools

import jax
import jax.numpy as jnp
from jax import lax
from jax.experimental import pallas as pl
from jax.experimental.pallas import tpu as pltpu
from jax.experimental.pallas import tpu_sc as plsc

N = 10000
E = 320000
D_IN = 128
D_HID = 128
D_OUT = 64
DQ = 32

NC = 2
NS = 16
NW = NC * NS
CH = 128
GB = 4
N_CHUNK = 2560
E_PAD = N_CHUNK * CH
CPW = N_CHUNK // NW
DUMP = 10047
N_HIST = 10240
N_ACC = 2512
WBR = 160

_mesh2 = plsc.VectorSubcoreMesh(
    core_axis_name="c", subcore_axis_name="s", num_cores=NC, num_subcores=NS)


def _wb_base(s):
    return jnp.minimum(s * WBR, N_ACC - WBR)


@functools.partial(
    pl.kernel,
    out_type=jax.ShapeDtypeStruct((NW, N_HIST), jnp.float32),
    mesh=_mesh2,
    scratch_types=[
        pltpu.VMEM((GB, CH), jnp.int32),
        pltpu.VMEM((N_HIST,), jnp.float32),
    ],
    compiler_params=pltpu.CompilerParams(needs_layout_passes=False),
)
def _deg_kernel(dst2d, out, idx_v, hist_v):
    c = lax.axis_index("c")
    s = lax.axis_index("s")
    wid = s * NC + c

    @pl.loop(0, N_HIST // 16)
    def _(r):
        hist_v[pl.ds(r * 16, 16)] = jnp.zeros((16,), jnp.float32)

    ones = jnp.ones((16,), jnp.float32)
    wbase = wid * CPW

    @pl.loop(0, CPW // GB)
    def _(g):
        pltpu.sync_copy(dst2d.at[pl.ds(wbase + g * GB, GB)], idx_v)
        for b in range(GB):
            for j in range(CH // 16):
                d = idx_v[b, pl.ds(j * 16, 16)]
                plsc.addupdate_scatter(hist_v, [d], ones)

    pltpu.sync_copy(hist_v, out.at[wid])


NBUF = 4


def _make_agg(nphase):
  @functools.partial(
      pl.kernel,
      out_type=jax.ShapeDtypeStruct((nphase, NC, N_ACC, D_HID), jnp.float32),
      mesh=_mesh2,
      scratch_types=[
          pltpu.VMEM((GB, CH), jnp.int32),
          pltpu.VMEM((GB, CH), jnp.int32),
          pltpu.VMEM((CPW, CH), jnp.int32),
          pltpu.VMEM((CPW, CH), jnp.int32),
          pltpu.VMEM((NBUF, CH, D_HID), jnp.float32),
          pltpu.VMEM((WBR, D_HID), jnp.float32),
          pltpu.VMEM_SHARED((N_ACC, D_HID), jnp.float32),
      ] + [pltpu.SemaphoreType.DMA] * (2 * NBUF),
      compiler_params=pltpu.CompilerParams(needs_layout_passes=False),
  )
  def _agg(src2d, dst2d, *rest):
    tables = rest[:nphase]
    parts = rest[nphase]
    (sidx_v, didx_v, gsrc_a, gdst_a, rbuf, wb_v, acc) = rest[
        nphase + 1:nphase + 8]
    sems = rest[nphase + 8:]
    sg = sems[:NBUF]
    ss = sems[NBUF:]
    c = lax.axis_index("c")
    s = lax.axis_index("s")
    wid = s * NC + c
    base = _wb_base(s)
    wbase = wid * CPW

    @pl.loop(0, CPW // GB)
    def _(g):
        pltpu.sync_copy(src2d.at[pl.ds(wbase + g * GB, GB)], sidx_v)
        pltpu.sync_copy(dst2d.at[pl.ds(wbase + g * GB, GB)], didx_v)
        for b in range(GB):
            for j in range(CH // 16):
                sl = pl.ds(j * 16, 16)
                sv = sidx_v[b, sl]
                dv = didx_v[b, sl]
                gsrc_a[g * GB + b, sl] = sv + (dv & 3) * N
                gdst_a[g * GB + b, sl] = lax.shift_right_logical(dv, 2)

    def fill_zeros():
        @pl.loop(0, WBR)
        def _(r):
            for j in range(D_HID // 16):
                wb_v[r, pl.ds(j * 16, 16)] = jnp.zeros((16,), jnp.float32)

    fill_zeros()
    for p in range(nphase):
        pltpu.sync_copy(wb_v, acc.at[pl.ds(base, WBR)])
        plsc.subcore_barrier()

        def fire_gather(t, buf):
            pltpu.async_copy(tables[p].at[gsrc_a.at[t]], rbuf.at[buf],
                             sg[buf])

        def wait_gather(buf):
            pltpu.make_async_copy(tables[p].at[gsrc_a.at[0]], rbuf.at[buf],
                                  sg[buf]).wait()

        def fire_scatter(t, buf):
            pltpu.async_copy(rbuf.at[buf], acc.at[gdst_a.at[t]], ss[buf],
                             add=True)

        def wait_scatter(buf):
            pltpu.make_async_copy(rbuf.at[buf], acc.at[gdst_a.at[0]],
                                  ss[buf]).wait()

        for w in range(NBUF - 1):
            fire_gather(w, w)

        @pl.loop(0, CPW)
        def _(t):
            def body(buf):
                nb = (buf + NBUF - 1) % NBUF

                @pl.when(t + NBUF - 1 < CPW)
                def _():
                    @pl.when(t >= 1)
                    def _():
                        wait_scatter(nb)

                    fire_gather(t + NBUF - 1, nb)

                wait_gather(buf)
                fire_scatter(t, buf)

            for m in range(NBUF):
                @pl.when(t % NBUF == m)
                def _(m=m):
                    body(m)

        for w in range(NBUF):
            wait_scatter(w)
        plsc.subcore_barrier()
        pltpu.sync_copy(acc.at[pl.ds(base, WBR)], wb_v)
        pltpu.sync_copy(wb_v, parts.at[p, c, pl.ds(base, WBR)])
        if p + 1 < nphase:
            fill_zeros()
            plsc.subcore_barrier()

  return _agg


_agg1 = _make_agg(4)
_agg2 = _make_agg(2)

_BN = 400
_GRID = N // _BN
_DC = 1280


def _tc0_body(d_b, o_b):
    ones = jnp.ones((NW, D_HID), jnp.float32)
    deg = lax.dot_general(d_b[...], ones, (((0,), (0,)), ((), ())),
                          preferred_element_type=jnp.float32)
    o_b[...] = 1.0 / jnp.sqrt(deg + 1.0)


_tc0 = pl.pallas_call(
    _tc0_body,
    grid=(N_HIST // _DC,),
    in_specs=[pl.BlockSpec((NW, _DC), lambda i: (0, i))],
    out_specs=pl.BlockSpec((_DC, D_HID), lambda i: (i, 0)),
    out_shape=jax.ShapeDtypeStruct((N_HIST, D_HID), jnp.float32),
)


def _tc1_body(x_b, w_b, d_b, o_b):
    h = jnp.dot(x_b[...], w_b[...], preferred_element_type=jnp.float32)
    o_b[...] = h * d_b[...]


def _tc2_body(a_b, h_b, d_b, d64_b, w_b, b1_b, o_b):
    pre = (a_b[0] + a_b[1] + h_b[...]) * d_b[...] + b1_b[...]
    h2 = jnp.maximum(pre, 0.0)
    o_b[...] = jnp.dot(
        h2, w_b[...], preferred_element_type=jnp.float32) * d64_b[...]


def _tc3_body(a_b, h_b, d64_b, b2_b, o_b):
    o = (a_b[0] + a_b[1] + h_b[...]) * d64_b[...] + b2_b[...]
    m = jnp.max(o, axis=1, keepdims=True)
    lse = jnp.log(jnp.sum(jnp.exp(o - m), axis=1, keepdims=True)) + m
    o_b[...] = o - lse


def _row_spec(d):
    return pl.BlockSpec((_BN, d), lambda i: (i, 0))


def _agg_spec(d):
    return pl.BlockSpec((NC, _BN, d), lambda i: (0, i, 0))


def _full_spec(shape):
    nd = len(shape)
    return pl.BlockSpec(shape, lambda i: (0,) * nd)


_tc1 = pl.pallas_call(
    _tc1_body,
    grid=(_GRID,),
    in_specs=[_row_spec(D_IN), _full_spec((D_IN, D_HID)), _row_spec(D_HID)],
    out_specs=_row_spec(D_HID),
    out_shape=jax.ShapeDtypeStruct((N, D_HID), jnp.float32),
)

_tc2 = pl.pallas_call(
    _tc2_body,
    grid=(_GRID,),
    in_specs=[
        _agg_spec(D_HID), _row_spec(D_HID), _row_spec(D_HID),
        _row_spec(D_OUT),
        _full_spec((D_HID, D_OUT)), _full_spec((1, D_HID)),
    ],
    out_specs=_row_spec(D_OUT),
    out_shape=jax.ShapeDtypeStruct((N, D_OUT), jnp.float32),
)

_tc3 = pl.pallas_call(
    _tc3_body,
    grid=(_GRID,),
    in_specs=[
        _agg_spec(D_OUT), _row_spec(D_OUT), _row_spec(D_OUT),
        _full_spec((1, D_OUT)),
    ],
    out_specs=_row_spec(D_OUT),
    out_shape=jax.ShapeDtypeStruct((N, D_OUT), jnp.float32),
)


def _stack_tables(h, nq):
    tabs = []
    for q in range(nq):
        hq = h[:, q * DQ:(q + 1) * DQ]
        planes = jnp.zeros((4, N, D_HID), jnp.float32)
        for k in range(4):
            planes = jax.lax.dynamic_update_slice(
                planes, hq[None], (k, 0, k * DQ))
        tabs.append(planes.reshape(4 * N, D_HID))
    return tabs


def _unpack_parts(parts, nq):
    pr = parts.reshape(nq, NC, 4 * N_ACC, DQ)
    return jnp.concatenate([pr[q] for q in range(nq)], axis=-1)


def kernel(x, edge_index, mask, W1, b1, W2, b2):
    del mask
    src = edge_index[0]
    dst = edge_index[1]
    pad = E_PAD - E
    src2d = jnp.concatenate([src, jnp.zeros((pad,), jnp.int32)]).reshape(
        N_CHUNK, CH)
    dst2d = jnp.concatenate([dst, jnp.full((pad,), DUMP, jnp.int32)]).reshape(
        N_CHUNK, CH)

    deg_t = _deg_kernel(dst2d)
    dis2d = _tc0(deg_t)
    dis64 = dis2d[:, :D_OUT]
    h1p = _tc1(x, W1, dis2d)
    parts1 = _agg1(src2d, dst2d, *_stack_tables(h1p, 4))
    agg1t = _unpack_parts(parts1, 4)
    h2w = _tc2(agg1t, h1p, dis2d, dis64, W2, b1.reshape(1, D_HID))
    parts2 = _agg2(src2d, dst2d, *_stack_tables(h2w, 2))
    agg2t = _unpack_parts(parts2, 2)
    return _tc3(agg2t, h2w, dis64, b2.reshape(1, D_OUT))
